# baseline (speedup 1.0000x reference)
import jax
import jax.numpy as jnp
import numpy as np
from jax import lax
from jax.experimental import pallas as pl
from jax.experimental.pallas import tpu as pltpu
from jax.experimental.pallas import tpu_sc as plsc

_VOCAB = 100000
_B = 1024
_L = 200
_BIAS = float(np.log(12000 / 10000))

_NC = 2
_NS = 16
_RPC = _B // _NC
_WIN = 16
_NV = _RPC // 16


def _sc_body(idx_hbm, r_hbm, out_hbm, r_sh, acc_sh, idx2_v, idx1_v, vals1_v,
             acc_v, io_v, sem):
    sid = lax.axis_index("s")
    cid = lax.axis_index("c")

    @pl.when(sid == 0)
    def _():
        pltpu.sync_copy(r_hbm, r_sh)

    a = (25 * sid // 16) * 8
    a_next = (25 * (sid + 1) // 16) * 8
    own = a_next - a
    a = pl.multiple_of(a, 8)
    col0 = pl.multiple_of(cid * _RPC, 128)
    pltpu.sync_copy(idx_hbm.at[pl.ds(a, _WIN), pl.ds(col0, _RPC)], idx2_v)

    zero16 = jnp.zeros((16,), jnp.float32)

    def zero_slot(k, _):
        acc_v[pl.ds(k * 16, 16)] = zero16
        return 0

    lax.fori_loop(0, _NV, zero_slot, 0)

    @pl.when(sid == 0)
    def _():
        pltpu.sync_copy(acc_v, acc_sh)

        def bias_slot(k, _):
            acc_v[pl.ds(k * 16, 16)] = jnp.full((16,), _BIAS, jnp.float32)
            return 0

        lax.fori_loop(0, _NV, bias_slot, 0)

    iota16 = lax.iota(jnp.int32, 16)

    def io_slot(k, _):
        io_v[pl.ds(k * 16, 16)] = iota16 + k * 16
        return 0

    lax.fori_loop(0, _NV, io_slot, 0)

    plsc.subcore_barrier()

    def fire(t, _):
        base = t * _RPC
        for k in range(_NV):
            v = idx2_v[t, pl.ds(k * 16, 16)]
            v = jnp.minimum(jnp.maximum(v, 0), _VOCAB - 1)
            idx1_v[pl.ds(base + k * 16, 16)] = v
        pltpu.async_copy(r_sh.at[idx1_v.at[pl.ds(base, _RPC)]],
                         vals1_v.at[pl.ds(base, _RPC)], sem)
        return 0

    lax.fori_loop(0, own, fire, 0)

    def row_step(t, _):
        pltpu.make_async_copy(r_hbm.at[pl.ds(0, _RPC)],
                              vals1_v.at[pl.ds(0, _RPC)], sem).wait()
        base = t * _RPC
        for k in range(_NV):
            plsc.addupdate(acc_v.at[pl.ds(k * 16, 16)],
                           vals1_v[pl.ds(base + k * 16, 16)])
        return 0

    lax.fori_loop(0, own, row_step, 0)

    pltpu.sync_copy(acc_v, acc_sh.at[io_v], add=True)
    plsc.subcore_barrier()

    @pl.when(sid == 0)
    def _():
        pltpu.sync_copy(acc_sh, out_hbm.at[pl.ds(col0, _RPC)])


@jax.jit
def _run(idx_t, r):
    mesh = plsc.VectorSubcoreMesh(core_axis_name="c", subcore_axis_name="s")
    return pl.kernel(
        _sc_body,
        mesh=mesh,
        compiler_params=pltpu.CompilerParams(needs_layout_passes=False),
        out_type=jax.ShapeDtypeStruct((_B,), jnp.float32),
        scratch_types=[
            pltpu.VMEM_SHARED((_VOCAB,), jnp.float32),
            pltpu.VMEM_SHARED((_RPC,), jnp.float32),
            pltpu.VMEM((_WIN, _RPC), jnp.int32),
            pltpu.VMEM((_WIN * _RPC,), jnp.int32),
            pltpu.VMEM((_WIN * _RPC,), jnp.float32),
            pltpu.VMEM((_RPC,), jnp.float32),
            pltpu.VMEM((_RPC,), jnp.int32),
            pltpu.SemaphoreType.DMA,
        ],
    )(idx_t, r)


def kernel(batch, r):
    return _run(batch.astype(jnp.int32).T, r)


# speedup vs baseline: 1.1058x; 1.1058x over previous
import jax
import jax.numpy as jnp
import numpy as np
from jax import lax
from jax.experimental import pallas as pl
from jax.experimental.pallas import tpu as pltpu
from jax.experimental.pallas import tpu_sc as plsc

_VOCAB = 100000
_B = 1024
_L = 200
_BIAS = float(np.log(12000 / 10000))

_NC = 2
_NS = 16
_NW = _NC * _NS
_ROWS_PER_W = _B // _NW
_HR = _ROWS_PER_W // 2


def _sc_body(idx_hbm, r_hbm, out_hbm, r_sh, idx_v, vals0_v, vals1_v, out_v,
             sem0, sem1):
    sid = lax.axis_index("s")
    wid = sid * _NC + lax.axis_index("c")
    row0 = wid * _ROWS_PER_W

    @pl.when(sid == 0)
    def _():
        pltpu.sync_copy(r_hbm, r_sh)

    pltpu.sync_copy(idx_hbm.at[pl.ds(row0 * _L, _ROWS_PER_W * _L)], idx_v)

    plsc.subcore_barrier()

    half = _HR * _L
    cp0 = pltpu.async_copy(r_sh.at[idx_v.at[pl.ds(0, half)]], vals0_v, sem0)
    cp1 = pltpu.async_copy(r_sh.at[idx_v.at[pl.ds(half, half)]], vals1_v, sem1)

    row_off = lax.iota(jnp.int32, 16) * _L
    zero = jnp.zeros((16,), jnp.float32)
    init = (jnp.full((16,), _BIAS, jnp.float32), zero, zero, zero)

    def acc_half(vref):
        def body(l, abcd):
            a, b, c, d = abcd
            p = row_off + 4 * l
            return (a + plsc.load_gather(vref, [p]),
                    b + plsc.load_gather(vref, [p + 1]),
                    c + plsc.load_gather(vref, [p + 2]),
                    d + plsc.load_gather(vref, [p + 3]))
        a, b, c, d = plsc.parallel_loop(0, _L // 4, carry=init, unroll=4)(body)
        return (a + b) + (c + d)

    cp0.wait()
    a0 = acc_half(vals0_v)
    cp1.wait()
    a1 = acc_half(vals1_v)
    out_v[pl.ds(0, 16)] = a0
    out_v[pl.ds(16, 16)] = a1
    pltpu.sync_copy(out_v, out_hbm.at[pl.ds(row0, _ROWS_PER_W)])


@jax.jit
def _run(idx2d, r):
    mesh = plsc.VectorSubcoreMesh(core_axis_name="c", subcore_axis_name="s")
    return pl.kernel(
        _sc_body,
        mesh=mesh,
        compiler_params=pltpu.CompilerParams(
            needs_layout_passes=False, skip_device_barrier=True
        ),
        out_type=jax.ShapeDtypeStruct((_B,), jnp.float32),
        scratch_types=[
            pltpu.VMEM_SHARED((_VOCAB,), jnp.float32),
            pltpu.VMEM((_ROWS_PER_W * _L,), jnp.int32),
            pltpu.VMEM((_HR * _L,), jnp.float32),
            pltpu.VMEM((_HR * _L,), jnp.float32),
            pltpu.VMEM((_ROWS_PER_W,), jnp.float32),
            pltpu.SemaphoreType.DMA,
            pltpu.SemaphoreType.DMA,
        ],
    )(idx2d, r)


def kernel(batch, r):
    return _run(batch.astype(jnp.int32).reshape(-1), r)


# speedup vs baseline: 1.1124x; 1.0060x over previous
import jax
import jax.numpy as jnp
import numpy as np
from jax import lax
from jax.experimental import pallas as pl
from jax.experimental.pallas import tpu as pltpu
from jax.experimental.pallas import tpu_sc as plsc

_VOCAB = 100000
_B = 1024
_L = 200
_BIAS = float(np.log(12000 / 10000))

_NC = 2
_NS = 16
_NW = _NC * _NS
_ROWS_PER_W = _B // _NW
_HR = _ROWS_PER_W // 2


def _sc_body(idx_hbm, r_hbm, out_hbm, r_sh, idx_v, vals0_v, vals1_v, out_v,
             sem0, sem1):
    sid = lax.axis_index("s")
    wid = sid * _NC + lax.axis_index("c")
    row0 = wid * _ROWS_PER_W

    @pl.when(sid == 0)
    def _():
        pltpu.sync_copy(r_hbm, r_sh)

    pltpu.sync_copy(idx_hbm.at[pl.ds(row0 * _L, _ROWS_PER_W * _L)], idx_v)

    plsc.subcore_barrier()

    half = _HR * _L
    cp0 = pltpu.async_copy(r_sh.at[idx_v.at[pl.ds(0, half)]], vals0_v, sem0)
    cp1 = pltpu.async_copy(r_sh.at[idx_v.at[pl.ds(half, half)]], vals1_v, sem1)

    row_off = lax.iota(jnp.int32, 16) * _L
    init = (jnp.full((16,), _BIAS, jnp.float32), jnp.zeros((16,), jnp.float32))

    def acc_half(vref):
        def body(l, ab):
            a, b = ab
            p = row_off + 2 * l
            return (a + plsc.load_gather(vref, [p]),
                    b + plsc.load_gather(vref, [p + 1]))
        a, b = plsc.parallel_loop(0, _L // 2, carry=init, unroll=4)(body)
        return a + b

    cp0.wait()
    a0 = acc_half(vals0_v)
    cp1.wait()
    a1 = acc_half(vals1_v)
    out_v[pl.ds(0, 16)] = a0
    out_v[pl.ds(16, 16)] = a1
    pltpu.sync_copy(out_v, out_hbm.at[pl.ds(row0, _ROWS_PER_W)])


@jax.jit
def _run(idx2d, r):
    mesh = plsc.VectorSubcoreMesh(core_axis_name="c", subcore_axis_name="s")
    return pl.kernel(
        _sc_body,
        mesh=mesh,
        compiler_params=pltpu.CompilerParams(
            needs_layout_passes=False, skip_device_barrier=True
        ),
        out_type=jax.ShapeDtypeStruct((_B,), jnp.float32),
        scratch_types=[
            pltpu.VMEM_SHARED((_VOCAB,), jnp.float32),
            pltpu.VMEM((_ROWS_PER_W * _L,), jnp.int32),
            pltpu.VMEM((_HR * _L,), jnp.float32),
            pltpu.VMEM((_HR * _L,), jnp.float32),
            pltpu.VMEM((_ROWS_PER_W,), jnp.float32),
            pltpu.SemaphoreType.DMA,
            pltpu.SemaphoreType.DMA,
        ],
    )(idx2d, r)


def kernel(batch, r):
    return _run(batch.astype(jnp.int32).reshape(-1), r)
